# Initial kernel scaffold; baseline (speedup 1.0000x reference)
import jax
import jax.numpy as jnp
from jax.experimental import pallas as pl


def kernel(x0, edge_index0, x1, edge_index1, sel0, sel1, Ws1_0, Wn1_0, b1_0, Ws2_0, Wn2_0, b2_0, Ws1_1, Wn1_1, b1_1, Ws2_1, Wn2_1, b2_1, W_attn):
    raise NotImplementedError("write your pallas kernel here")


# speedup vs baseline: 2.0180x; 2.0180x over previous
import functools

import jax
import jax.numpy as jnp
from jax import lax
from jax.experimental import pallas as pl
from jax.experimental.pallas import tpu as pltpu
from jax.experimental.pallas import tpu_sc as plsc

N = 10000
E = 320000
D = 128
B = 1024
C = 8

NC = 2
NS = 16
L = 128

T = -(-E // (NS * L * 8)) * 8
EPAD = NS * T * L
EROWS = NS * T

ZR = -(-(N + 1) // (NS * 8)) * 8
NACC = ZR * NS
IB = 8


def _seg_sum_body(x0, src0, dst0, x1, src1, dst1, zrow,
                  agg0, agg1,
                  acc, src_v, dst_v, rows_v, sem):
  c = lax.axis_index("c")
  s = lax.axis_index("s")

  pltpu.sync_copy(zrow, acc.at[pl.ds(s * ZR, ZR)])
  plsc.subcore_barrier()

  def work(x_hbm, src_hbm, dst_hbm):
    base = s * T

    @pl.loop(0, T // IB)
    def _(ob):
      start = pl.multiple_of(base + ob * IB, 8)
      pltpu.sync_copy(src_hbm.at[pl.ds(start, IB)], src_v)
      pltpu.sync_copy(dst_hbm.at[pl.ds(start, IB)], dst_v)

      for j in range(IB):
        pltpu.async_copy(x_hbm.at[src_v.at[j]], rows_v, sem).wait()
        pltpu.sync_copy(rows_v, acc.at[dst_v.at[j]], add=True)

  @pl.when(c == 0)
  def _():
    work(x0, src0, dst0)

  @pl.when(c == 1)
  def _():
    work(x1, src1, dst1)

  plsc.subcore_barrier()

  @pl.when(c == 0)
  def _():
    pltpu.sync_copy(acc.at[pl.ds(s * ZR, ZR)], agg0.at[pl.ds(s * ZR, ZR)])

  @pl.when(c == 1)
  def _():
    pltpu.sync_copy(acc.at[pl.ds(s * ZR, ZR)], agg1.at[pl.ds(s * ZR, ZR)])


_seg_sum = pl.kernel(
    _seg_sum_body,
    out_type=[jax.ShapeDtypeStruct((NACC, D), jnp.float32),
              jax.ShapeDtypeStruct((NACC, D), jnp.float32)],
    mesh=plsc.VectorSubcoreMesh(core_axis_name="c", subcore_axis_name="s"),
    scratch_types=[pltpu.VMEM_SHARED((NACC, D), jnp.float32),
                   pltpu.VMEM((IB, L), jnp.int32),
                   pltpu.VMEM((IB, L), jnp.int32),
                   pltpu.VMEM((L, D), jnp.float32),
                   pltpu.SemaphoreType.DMA],
    name="sc_seg_sum")


def _gather_sel_body(emb0, emb1, sel0r, sel1r, out0, out1,
                     idx_v, rows_v, sem):
  c = lax.axis_index("c")
  s = lax.axis_index("s")
  per_sub = (B * C) // NS

  def work(emb, sel_flat, out):
    pltpu.sync_copy(sel_flat.at[pl.ds(s * per_sub, per_sub)], idx_v)
    for j in range(per_sub // L):
      r = pl.multiple_of((s * (per_sub // L) + j) * L, 8)
      pltpu.async_copy(emb.at[idx_v.at[pl.ds(j * L, L)]], rows_v, sem).wait()
      pltpu.sync_copy(rows_v, out.at[pl.ds(r, L)])

  @pl.when(c == 0)
  def _():
    work(emb0, sel0r, out0)

  @pl.when(c == 1)
  def _():
    work(emb1, sel1r, out1)


_gather_sel = pl.kernel(
    _gather_sel_body,
    out_type=[jax.ShapeDtypeStruct((B * C, D), jnp.float32),
              jax.ShapeDtypeStruct((B * C, D), jnp.float32)],
    mesh=plsc.VectorSubcoreMesh(core_axis_name="c", subcore_axis_name="s"),
    scratch_types=[pltpu.VMEM(((B * C) // NS,), jnp.int32),
                   pltpu.VMEM((L, D), jnp.float32),
                   pltpu.SemaphoreType.DMA],
    name="sc_gather_sel")


BN = 1000


def _layer_body(relu, x0, agg0, deg0, x1, agg1, deg1,
                ws0, wn0, b0, ws1, wn1, b1, h0, h1):
  def one(x_ref, agg_ref, deg_ref, ws, wn, b, out_ref):
    rdeg = 1.0 / jnp.maximum(deg_ref[:, 0:1], 1.0)
    hn = agg_ref[...] * rdeg
    out = (jnp.dot(x_ref[...], ws[...], preferred_element_type=jnp.float32)
           + jnp.dot(hn, wn[...], preferred_element_type=jnp.float32)
           + b[...])
    if relu:
      out = jnp.maximum(out, 0.0)
    out_ref[...] = out

  one(x0, agg0, deg0, ws0, wn0, b0, h0)
  one(x1, agg1, deg1, ws1, wn1, b1, h1)


def _make_layer(relu):
  row = pl.BlockSpec((BN, D), lambda i: (i, 0))
  degc = pl.BlockSpec((BN, D), lambda i: (i, 0))
  full = pl.BlockSpec((D, D), lambda i: (0, 0))
  bias = pl.BlockSpec((1, D), lambda i: (0, 0))
  return pl.pallas_call(
      functools.partial(_layer_body, relu),
      grid=(N // BN,),
      in_specs=[row, row, degc, row, row, degc, full, full, bias,
                full, full, bias],
      out_specs=[row, row],
      out_shape=[jax.ShapeDtypeStruct((N, D), jnp.float32),
                 jax.ShapeDtypeStruct((N, D), jnp.float32)],
      name="tc_sage_layer_relu" if relu else "tc_sage_layer")


_layer_relu = _make_layer(True)
_layer_lin = _make_layer(False)


BB = 64


def _attn_body(s0_ref, s1_ref, wa_ref, out_ref):
  wm = jnp.mean(wa_ref[...], axis=2)
  s0 = s0_ref[...]
  s1 = s1_ref[...]
  sc0 = jnp.sum(s0 * wm[:, None, :], axis=2)
  sc1 = jnp.sum(s1 * wm[:, None, :], axis=2)
  scores = jnp.concatenate([sc0, sc1], axis=1)
  m = jnp.max(scores, axis=1, keepdims=True)
  e = jnp.exp(scores - m)
  p = e / jnp.sum(e, axis=1, keepdims=True)
  out_ref[...] = (jnp.sum(s0 * p[:, :C, None], axis=1)
                  + jnp.sum(s1 * p[:, C:, None], axis=1))


_attn = pl.pallas_call(
    _attn_body,
    grid=(B // BB,),
    in_specs=[pl.BlockSpec((BB, C, D), lambda i: (i, 0, 0)),
              pl.BlockSpec((BB, C, D), lambda i: (i, 0, 0)),
              pl.BlockSpec((BB, D, D), lambda i: (i, 0, 0))],
    out_specs=pl.BlockSpec((BB, D), lambda i: (i, 0)),
    out_shape=jax.ShapeDtypeStruct((B, D), jnp.float32),
    name="tc_attn_combine")


def _pad_edges(edge_index):
  src = edge_index[0]
  dst = edge_index[1]
  pad = EPAD - E
  src = jnp.concatenate([src, jnp.zeros((pad,), jnp.int32)])
  dst = jnp.concatenate([dst, jnp.full((pad,), N, jnp.int32)])
  return src.reshape(EROWS, L), dst.reshape(EROWS, L)


def kernel(x0, edge_index0, x1, edge_index1, sel0, sel1,
           Ws1_0, Wn1_0, b1_0, Ws2_0, Wn2_0, b2_0,
           Ws1_1, Wn1_1, b1_1, Ws2_1, Wn2_1, b2_1,
           W_attn):
  src0, dst0 = _pad_edges(edge_index0)
  src1, dst1 = _pad_edges(edge_index1)
  zrow = jnp.zeros((ZR, D), jnp.float32)
  ones_tab = jnp.ones((NACC, D), jnp.float32)

  degw0, degw1 = _seg_sum(ones_tab, dst0, dst0, ones_tab, dst1, dst1, zrow)

  agg0, agg1 = _seg_sum(x0, src0, dst0, x1, src1, dst1, zrow)

  h0, h1 = _layer_relu(x0, agg0, degw0, x1, agg1, degw1,
                       Ws1_0, Wn1_0, b1_0.reshape(1, D),
                       Ws1_1, Wn1_1, b1_1.reshape(1, D))

  agg0b, agg1b = _seg_sum(h0, src0, dst0, h1, src1, dst1, zrow)

  emb0, emb1 = _layer_lin(h0, agg0b, degw0, h1, agg1b, degw1,
                          Ws2_0, Wn2_0, b2_0.reshape(1, D),
                          Ws2_1, Wn2_1, b2_1.reshape(1, D))

  g0, g1 = _gather_sel(emb0, emb1, sel0.reshape(B * C), sel1.reshape(B * C))

  return _attn(g0.reshape(B, C, D), g1.reshape(B, C, D), W_attn)


# speedup vs baseline: 3.2601x; 1.6155x over previous
import functools

import jax
import jax.numpy as jnp
from jax import lax
from jax.experimental import pallas as pl
from jax.experimental.pallas import tpu as pltpu
from jax.experimental.pallas import tpu_sc as plsc

N = 10000
E = 320000
D = 128
B = 1024
C = 8

NC = 2
NS = 16
L = 128

T = -(-E // (NS * L * 8)) * 8
EPAD = NS * T * L
EROWS = NS * T

ZR = -(-(N + 1) // (NS * 8)) * 8
NACC = ZR * NS
IB = 8


def _seg_sum_body(x0, src0, dst0, x1, src1, dst1, zrow,
                  agg0, agg1,
                  acc, src_v, dst_v, rows_a, rows_b,
                  gsem_a, gsem_b, ssem_a, ssem_b):
  c = lax.axis_index("c")
  s = lax.axis_index("s")
  bufs = (rows_a, rows_b)
  gsems = (gsem_a, gsem_b)
  ssems = (ssem_a, ssem_b)

  pltpu.sync_copy(zrow, acc.at[pl.ds(s * ZR, ZR)])
  plsc.subcore_barrier()

  def work(x_hbm, src_hbm, dst_hbm):
    base = s * T

    @pl.loop(0, T // IB)
    def _(ob):
      start = pl.multiple_of(base + ob * IB, 8)
      pltpu.sync_copy(src_hbm.at[pl.ds(start, IB)], src_v)
      pltpu.sync_copy(dst_hbm.at[pl.ds(start, IB)], dst_v)

      gcps = [None] * IB
      scps = [None] * IB
      for j in range(IB):
        b = j % 2
        if j >= 2:
          scps[j - 2].wait()
        gcps[j] = pltpu.async_copy(x_hbm.at[src_v.at[j]], bufs[b], gsems[b])
        if j >= 1:
          jj = j - 1
          gcps[jj].wait()
          scps[jj] = pltpu.async_copy(bufs[jj % 2], acc.at[dst_v.at[jj]],
                                      ssems[jj % 2], add=True)
      gcps[IB - 1].wait()
      scps[IB - 1] = pltpu.async_copy(bufs[(IB - 1) % 2],
                                      acc.at[dst_v.at[IB - 1]],
                                      ssems[(IB - 1) % 2], add=True)
      scps[IB - 2].wait()
      scps[IB - 1].wait()

  @pl.when(c == 0)
  def _():
    work(x0, src0, dst0)

  @pl.when(c == 1)
  def _():
    work(x1, src1, dst1)

  plsc.subcore_barrier()

  @pl.when(c == 0)
  def _():
    pltpu.sync_copy(acc.at[pl.ds(s * ZR, ZR)], agg0.at[pl.ds(s * ZR, ZR)])

  @pl.when(c == 1)
  def _():
    pltpu.sync_copy(acc.at[pl.ds(s * ZR, ZR)], agg1.at[pl.ds(s * ZR, ZR)])


_seg_sum = pl.kernel(
    _seg_sum_body,
    out_type=[jax.ShapeDtypeStruct((NACC, D), jnp.float32),
              jax.ShapeDtypeStruct((NACC, D), jnp.float32)],
    mesh=plsc.VectorSubcoreMesh(core_axis_name="c", subcore_axis_name="s"),
    scratch_types=[pltpu.VMEM_SHARED((NACC, D), jnp.float32),
                   pltpu.VMEM((IB, L), jnp.int32),
                   pltpu.VMEM((IB, L), jnp.int32),
                   pltpu.VMEM((L, D), jnp.float32),
                   pltpu.VMEM((L, D), jnp.float32),
                   pltpu.SemaphoreType.DMA,
                   pltpu.SemaphoreType.DMA,
                   pltpu.SemaphoreType.DMA,
                   pltpu.SemaphoreType.DMA],
    name="sc_seg_sum")


def _deg_sum_body(dst0, dst1, zrow, onesrow,
                  deg0, deg1,
                  acc, dst_v, rows_v, sem):
  c = lax.axis_index("c")
  s = lax.axis_index("s")

  pltpu.sync_copy(zrow, acc.at[pl.ds(s * ZR, ZR)])
  pltpu.sync_copy(onesrow, rows_v)
  plsc.subcore_barrier()

  def work(dst_hbm):
    base = s * T

    @pl.loop(0, T // IB)
    def _(ob):
      start = pl.multiple_of(base + ob * IB, 8)
      pltpu.sync_copy(dst_hbm.at[pl.ds(start, IB)], dst_v)
      cps = [pltpu.async_copy(rows_v, acc.at[dst_v.at[j]], sem, add=True)
             for j in range(IB)]
      for cp in cps:
        cp.wait()

  @pl.when(c == 0)
  def _():
    work(dst0)

  @pl.when(c == 1)
  def _():
    work(dst1)

  plsc.subcore_barrier()

  @pl.when(c == 0)
  def _():
    pltpu.sync_copy(acc.at[pl.ds(s * ZR, ZR)], deg0.at[pl.ds(s * ZR, ZR)])

  @pl.when(c == 1)
  def _():
    pltpu.sync_copy(acc.at[pl.ds(s * ZR, ZR)], deg1.at[pl.ds(s * ZR, ZR)])


_deg_sum = pl.kernel(
    _deg_sum_body,
    out_type=[jax.ShapeDtypeStruct((NACC, D), jnp.float32),
              jax.ShapeDtypeStruct((NACC, D), jnp.float32)],
    mesh=plsc.VectorSubcoreMesh(core_axis_name="c", subcore_axis_name="s"),
    scratch_types=[pltpu.VMEM_SHARED((NACC, D), jnp.float32),
                   pltpu.VMEM((IB, L), jnp.int32),
                   pltpu.VMEM((L, D), jnp.float32),
                   pltpu.SemaphoreType.DMA],
    name="sc_deg_sum")


def _gather_sel_body(emb0, emb1, sel0r, sel1r, out0, out1,
                     idx_v, rows_v, sem):
  c = lax.axis_index("c")
  s = lax.axis_index("s")
  per_sub = (B * C) // NS

  def work(emb, sel_flat, out):
    pltpu.sync_copy(sel_flat.at[pl.ds(s * per_sub, per_sub)], idx_v)
    for j in range(per_sub // L):
      r = pl.multiple_of((s * (per_sub // L) + j) * L, 8)
      pltpu.async_copy(emb.at[idx_v.at[pl.ds(j * L, L)]], rows_v, sem).wait()
      pltpu.sync_copy(rows_v, out.at[pl.ds(r, L)])

  @pl.when(c == 0)
  def _():
    work(emb0, sel0r, out0)

  @pl.when(c == 1)
  def _():
    work(emb1, sel1r, out1)


_gather_sel = pl.kernel(
    _gather_sel_body,
    out_type=[jax.ShapeDtypeStruct((B * C, D), jnp.float32),
              jax.ShapeDtypeStruct((B * C, D), jnp.float32)],
    mesh=plsc.VectorSubcoreMesh(core_axis_name="c", subcore_axis_name="s"),
    scratch_types=[pltpu.VMEM(((B * C) // NS,), jnp.int32),
                   pltpu.VMEM((L, D), jnp.float32),
                   pltpu.SemaphoreType.DMA],
    name="sc_gather_sel")


BN = 1000


def _layer_body(relu, x0, agg0, deg0, x1, agg1, deg1,
                ws0, wn0, b0, ws1, wn1, b1, h0, h1):
  def one(x_ref, agg_ref, deg_ref, ws, wn, b, out_ref):
    rdeg = 1.0 / jnp.maximum(deg_ref[:, 0:1], 1.0)
    hn = agg_ref[...] * rdeg
    out = (jnp.dot(x_ref[...], ws[...], preferred_element_type=jnp.float32)
           + jnp.dot(hn, wn[...], preferred_element_type=jnp.float32)
           + b[...])
    if relu:
      out = jnp.maximum(out, 0.0)
    out_ref[...] = out

  one(x0, agg0, deg0, ws0, wn0, b0, h0)
  one(x1, agg1, deg1, ws1, wn1, b1, h1)


def _make_layer(relu):
  row = pl.BlockSpec((BN, D), lambda i: (i, 0))
  degc = pl.BlockSpec((BN, D), lambda i: (i, 0))
  full = pl.BlockSpec((D, D), lambda i: (0, 0))
  bias = pl.BlockSpec((1, D), lambda i: (0, 0))
  return pl.pallas_call(
      functools.partial(_layer_body, relu),
      grid=(N // BN,),
      in_specs=[row, row, degc, row, row, degc, full, full, bias,
                full, full, bias],
      out_specs=[row, row],
      out_shape=[jax.ShapeDtypeStruct((N, D), jnp.float32),
                 jax.ShapeDtypeStruct((N, D), jnp.float32)],
      name="tc_sage_layer_relu" if relu else "tc_sage_layer")


_layer_relu = _make_layer(True)
_layer_lin = _make_layer(False)


BB = 64


def _attn_body(s0_ref, s1_ref, wa_ref, out_ref):
  wm = jnp.mean(wa_ref[...], axis=2)
  s0 = s0_ref[...]
  s1 = s1_ref[...]
  sc0 = jnp.sum(s0 * wm[:, None, :], axis=2)
  sc1 = jnp.sum(s1 * wm[:, None, :], axis=2)
  scores = jnp.concatenate([sc0, sc1], axis=1)
  m = jnp.max(scores, axis=1, keepdims=True)
  e = jnp.exp(scores - m)
  p = e / jnp.sum(e, axis=1, keepdims=True)
  out_ref[...] = (jnp.sum(s0 * p[:, :C, None], axis=1)
                  + jnp.sum(s1 * p[:, C:, None], axis=1))


_attn = pl.pallas_call(
    _attn_body,
    grid=(B // BB,),
    in_specs=[pl.BlockSpec((BB, C, D), lambda i: (i, 0, 0)),
              pl.BlockSpec((BB, C, D), lambda i: (i, 0, 0)),
              pl.BlockSpec((BB, D, D), lambda i: (i, 0, 0))],
    out_specs=pl.BlockSpec((BB, D), lambda i: (i, 0)),
    out_shape=jax.ShapeDtypeStruct((B, D), jnp.float32),
    name="tc_attn_combine")


def _pad_edges(edge_index):
  src = edge_index[0]
  dst = edge_index[1]
  pad = EPAD - E
  src = jnp.concatenate([src, jnp.zeros((pad,), jnp.int32)])
  dst = jnp.concatenate([dst, jnp.full((pad,), N, jnp.int32)])
  return src.reshape(EROWS, L), dst.reshape(EROWS, L)


def kernel(x0, edge_index0, x1, edge_index1, sel0, sel1,
           Ws1_0, Wn1_0, b1_0, Ws2_0, Wn2_0, b2_0,
           Ws1_1, Wn1_1, b1_1, Ws2_1, Wn2_1, b2_1,
           W_attn):
  src0, dst0 = _pad_edges(edge_index0)
  src1, dst1 = _pad_edges(edge_index1)
  zrow = jnp.zeros((ZR, D), jnp.float32)
  onesrow = jnp.ones((L, D), jnp.float32)

  degw0, degw1 = _deg_sum(dst0, dst1, zrow, onesrow)

  agg0, agg1 = _seg_sum(x0, src0, dst0, x1, src1, dst1, zrow)

  h0, h1 = _layer_relu(x0, agg0, degw0, x1, agg1, degw1,
                       Ws1_0, Wn1_0, b1_0.reshape(1, D),
                       Ws1_1, Wn1_1, b1_1.reshape(1, D))

  agg0b, agg1b = _seg_sum(h0, src0, dst0, h1, src1, dst1, zrow)

  emb0, emb1 = _layer_lin(h0, agg0b, degw0, h1, agg1b, degw1,
                          Ws2_0, Wn2_0, b2_0.reshape(1, D),
                          Ws2_1, Wn2_1, b2_1.reshape(1, D))

  g0, g1 = _gather_sel(emb0, emb1, sel0.reshape(B * C), sel1.reshape(B * C))

  return _attn(g0.reshape(B, C, D), g1.reshape(B, C, D), W_attn)


# speedup vs baseline: 3.2944x; 1.0105x over previous
import functools

import jax
import jax.numpy as jnp
from jax import lax
from jax.experimental import pallas as pl
from jax.experimental.pallas import tpu as pltpu
from jax.experimental.pallas import tpu_sc as plsc

N = 10000
E = 320000
D = 128
B = 1024
C = 8

NC = 2
NS = 16
L = 128

T = -(-E // (NS * L * 8)) * 8
EPAD = NS * T * L
EROWS = NS * T

ZR = -(-(N + 1) // (NS * 8)) * 8
NACC = ZR * NS
IB = 8


H = L // 2


def _seg_sum_body(x0, src0, dst0, x1, src1, dst1, zrow,
                  agg0, agg1,
                  acc, src_va, dst_va, src_vb, dst_vb, ring,
                  gsem0, gsem1, gsem2, gsem3, ssem0, ssem1):
  c = lax.axis_index("c")
  s = lax.axis_index("s")
  gsems = (gsem0, gsem1, gsem2, gsem3)
  ssems = (ssem0, ssem1)

  pltpu.sync_copy(zrow, acc.at[pl.ds(s * ZR, ZR)])
  plsc.subcore_barrier()

  def work(x_hbm, src_hbm, dst_hbm):
    base = s * T

    def scatter(j, dst_v):
      q0 = (2 * j) % 4
      return pltpu.async_copy(ring.at[pl.ds(q0 * H, L)],
                              acc.at[dst_v.at[j]], ssems[j % 2], add=True)

    def absorb(j, dst_v):
      pltpu.make_async_copy(ring.at[pl.ds(0, L)], acc.at[dst_v.at[j]],
                            ssems[j % 2]).wait()

    def do_ob(ob, src_v, dst_v, first):
      start = pl.multiple_of(base + ob * IB, 8)
      pltpu.sync_copy(src_hbm.at[pl.ds(start, IB)], src_v)
      pltpu.sync_copy(dst_hbm.at[pl.ds(start, IB)], dst_v)
      gcps = [None] * IB
      scps = [None] * IB
      for j in range(IB):
        q0 = (2 * j) % 4
        if j < 2:
          if first is None:
            absorb(j, dst_v)
          else:
            @pl.when(first > 0)
            def _():
              absorb(j, dst_v)
        else:
          scps[j - 2].wait()
        gcps[j] = (
            pltpu.async_copy(x_hbm.at[src_v.at[j, pl.ds(0, H)]],
                             ring.at[pl.ds(q0 * H, H)], gsems[q0]),
            pltpu.async_copy(x_hbm.at[src_v.at[j, pl.ds(H, H)]],
                             ring.at[pl.ds((q0 + 1) * H, H)], gsems[q0 + 1]))
        if j >= 1:
          gcps[j - 1][0].wait()
          gcps[j - 1][1].wait()
          scps[j - 1] = scatter(j - 1, dst_v)
      gcps[IB - 1][0].wait()
      gcps[IB - 1][1].wait()
      scps[IB - 1] = scatter(IB - 1, dst_v)

    @pl.loop(0, T // IB // 2)
    def _(obb):
      do_ob(2 * obb, src_va, dst_va, obb)
      do_ob(2 * obb + 1, src_vb, dst_vb, None)

    pltpu.make_async_copy(ring.at[pl.ds(0, L)], acc.at[dst_vb.at[0]],
                          ssems[0]).wait()
    pltpu.make_async_copy(ring.at[pl.ds(0, L)], acc.at[dst_vb.at[1]],
                          ssems[1]).wait()

  @pl.when(c == 0)
  def _():
    work(x0, src0, dst0)

  @pl.when(c == 1)
  def _():
    work(x1, src1, dst1)

  plsc.subcore_barrier()

  @pl.when(c == 0)
  def _():
    pltpu.sync_copy(acc.at[pl.ds(s * ZR, ZR)], agg0.at[pl.ds(s * ZR, ZR)])

  @pl.when(c == 1)
  def _():
    pltpu.sync_copy(acc.at[pl.ds(s * ZR, ZR)], agg1.at[pl.ds(s * ZR, ZR)])


_seg_sum = pl.kernel(
    _seg_sum_body,
    out_type=[jax.ShapeDtypeStruct((NACC, D), jnp.float32),
              jax.ShapeDtypeStruct((NACC, D), jnp.float32)],
    mesh=plsc.VectorSubcoreMesh(core_axis_name="c", subcore_axis_name="s"),
    scratch_types=[pltpu.VMEM_SHARED((NACC, D), jnp.float32),
                   pltpu.VMEM((IB, L), jnp.int32),
                   pltpu.VMEM((IB, L), jnp.int32),
                   pltpu.VMEM((IB, L), jnp.int32),
                   pltpu.VMEM((IB, L), jnp.int32),
                   pltpu.VMEM((4 * H, D), jnp.float32),
                   pltpu.SemaphoreType.DMA,
                   pltpu.SemaphoreType.DMA,
                   pltpu.SemaphoreType.DMA,
                   pltpu.SemaphoreType.DMA,
                   pltpu.SemaphoreType.DMA,
                   pltpu.SemaphoreType.DMA],
    name="sc_seg_sum")


def _deg_sum_body(dst0, dst1, zrow, onesrow,
                  deg0, deg1,
                  acc, dst_v, rows_v, sem):
  c = lax.axis_index("c")
  s = lax.axis_index("s")

  pltpu.sync_copy(zrow, acc.at[pl.ds(s * ZR, ZR)])
  pltpu.sync_copy(onesrow, rows_v)
  plsc.subcore_barrier()

  def work(dst_hbm):
    base = s * T

    @pl.loop(0, T // IB)
    def _(ob):
      start = pl.multiple_of(base + ob * IB, 8)
      pltpu.sync_copy(dst_hbm.at[pl.ds(start, IB)], dst_v)
      cps = [pltpu.async_copy(rows_v, acc.at[dst_v.at[j]], sem, add=True)
             for j in range(IB)]
      for cp in cps:
        cp.wait()

  @pl.when(c == 0)
  def _():
    work(dst0)

  @pl.when(c == 1)
  def _():
    work(dst1)

  plsc.subcore_barrier()

  @pl.when(c == 0)
  def _():
    pltpu.sync_copy(acc.at[pl.ds(s * ZR, ZR)], deg0.at[pl.ds(s * ZR, ZR)])

  @pl.when(c == 1)
  def _():
    pltpu.sync_copy(acc.at[pl.ds(s * ZR, ZR)], deg1.at[pl.ds(s * ZR, ZR)])


_deg_sum = pl.kernel(
    _deg_sum_body,
    out_type=[jax.ShapeDtypeStruct((NACC, D), jnp.float32),
              jax.ShapeDtypeStruct((NACC, D), jnp.float32)],
    mesh=plsc.VectorSubcoreMesh(core_axis_name="c", subcore_axis_name="s"),
    scratch_types=[pltpu.VMEM_SHARED((NACC, D), jnp.float32),
                   pltpu.VMEM((IB, L), jnp.int32),
                   pltpu.VMEM((L, D), jnp.float32),
                   pltpu.SemaphoreType.DMA],
    name="sc_deg_sum")


def _gather_sel_body(emb0, emb1, sel0r, sel1r, out0, out1,
                     idx_v, rows_v, sem):
  c = lax.axis_index("c")
  s = lax.axis_index("s")
  per_sub = (B * C) // NS

  def work(emb, sel_flat, out):
    pltpu.sync_copy(sel_flat.at[pl.ds(s * per_sub, per_sub)], idx_v)
    for j in range(per_sub // L):
      r = pl.multiple_of((s * (per_sub // L) + j) * L, 8)
      pltpu.async_copy(emb.at[idx_v.at[pl.ds(j * L, L)]], rows_v, sem).wait()
      pltpu.sync_copy(rows_v, out.at[pl.ds(r, L)])

  @pl.when(c == 0)
  def _():
    work(emb0, sel0r, out0)

  @pl.when(c == 1)
  def _():
    work(emb1, sel1r, out1)


_gather_sel = pl.kernel(
    _gather_sel_body,
    out_type=[jax.ShapeDtypeStruct((B * C, D), jnp.float32),
              jax.ShapeDtypeStruct((B * C, D), jnp.float32)],
    mesh=plsc.VectorSubcoreMesh(core_axis_name="c", subcore_axis_name="s"),
    scratch_types=[pltpu.VMEM(((B * C) // NS,), jnp.int32),
                   pltpu.VMEM((L, D), jnp.float32),
                   pltpu.SemaphoreType.DMA],
    name="sc_gather_sel")


BN = 1000


def _layer_body(relu, x0, agg0, deg0, x1, agg1, deg1,
                ws0, wn0, b0, ws1, wn1, b1, h0, h1):
  def one(x_ref, agg_ref, deg_ref, ws, wn, b, out_ref):
    rdeg = 1.0 / jnp.maximum(deg_ref[:, 0:1], 1.0)
    hn = agg_ref[...] * rdeg
    out = (jnp.dot(x_ref[...], ws[...], preferred_element_type=jnp.float32)
           + jnp.dot(hn, wn[...], preferred_element_type=jnp.float32)
           + b[...])
    if relu:
      out = jnp.maximum(out, 0.0)
    out_ref[...] = out

  one(x0, agg0, deg0, ws0, wn0, b0, h0)
  one(x1, agg1, deg1, ws1, wn1, b1, h1)


def _make_layer(relu):
  row = pl.BlockSpec((BN, D), lambda i: (i, 0))
  degc = pl.BlockSpec((BN, D), lambda i: (i, 0))
  full = pl.BlockSpec((D, D), lambda i: (0, 0))
  bias = pl.BlockSpec((1, D), lambda i: (0, 0))
  return pl.pallas_call(
      functools.partial(_layer_body, relu),
      grid=(N // BN,),
      in_specs=[row, row, degc, row, row, degc, full, full, bias,
                full, full, bias],
      out_specs=[row, row],
      out_shape=[jax.ShapeDtypeStruct((N, D), jnp.float32),
                 jax.ShapeDtypeStruct((N, D), jnp.float32)],
      name="tc_sage_layer_relu" if relu else "tc_sage_layer")


_layer_relu = _make_layer(True)
_layer_lin = _make_layer(False)


BB = 64


def _attn_body(s0_ref, s1_ref, wa_ref, out_ref):
  wm = jnp.mean(wa_ref[...], axis=2)
  s0 = s0_ref[...]
  s1 = s1_ref[...]
  sc0 = jnp.sum(s0 * wm[:, None, :], axis=2)
  sc1 = jnp.sum(s1 * wm[:, None, :], axis=2)
  scores = jnp.concatenate([sc0, sc1], axis=1)
  m = jnp.max(scores, axis=1, keepdims=True)
  e = jnp.exp(scores - m)
  p = e / jnp.sum(e, axis=1, keepdims=True)
  out_ref[...] = (jnp.sum(s0 * p[:, :C, None], axis=1)
                  + jnp.sum(s1 * p[:, C:, None], axis=1))


_attn = pl.pallas_call(
    _attn_body,
    grid=(B // BB,),
    in_specs=[pl.BlockSpec((BB, C, D), lambda i: (i, 0, 0)),
              pl.BlockSpec((BB, C, D), lambda i: (i, 0, 0)),
              pl.BlockSpec((BB, D, D), lambda i: (i, 0, 0))],
    out_specs=pl.BlockSpec((BB, D), lambda i: (i, 0)),
    out_shape=jax.ShapeDtypeStruct((B, D), jnp.float32),
    name="tc_attn_combine")


def _pad_edges(edge_index):
  src = edge_index[0]
  dst = edge_index[1]
  pad = EPAD - E
  src = jnp.concatenate([src, jnp.zeros((pad,), jnp.int32)])
  dst = jnp.concatenate([dst, jnp.full((pad,), N, jnp.int32)])
  return src.reshape(EROWS, L), dst.reshape(EROWS, L)


def kernel(x0, edge_index0, x1, edge_index1, sel0, sel1,
           Ws1_0, Wn1_0, b1_0, Ws2_0, Wn2_0, b2_0,
           Ws1_1, Wn1_1, b1_1, Ws2_1, Wn2_1, b2_1,
           W_attn):
  src0, dst0 = _pad_edges(edge_index0)
  src1, dst1 = _pad_edges(edge_index1)
  zrow = jnp.zeros((ZR, D), jnp.float32)
  onesrow = jnp.ones((L, D), jnp.float32)

  degw0, degw1 = _deg_sum(dst0, dst1, zrow, onesrow)

  agg0, agg1 = _seg_sum(x0, src0, dst0, x1, src1, dst1, zrow)

  h0, h1 = _layer_relu(x0, agg0, degw0, x1, agg1, degw1,
                       Ws1_0, Wn1_0, b1_0.reshape(1, D),
                       Ws1_1, Wn1_1, b1_1.reshape(1, D))

  agg0b, agg1b = _seg_sum(h0, src0, dst0, h1, src1, dst1, zrow)

  emb0, emb1 = _layer_lin(h0, agg0b, degw0, h1, agg1b, degw1,
                          Ws2_0, Wn2_0, b2_0.reshape(1, D),
                          Ws2_1, Wn2_1, b2_1.reshape(1, D))

  g0, g1 = _gather_sel(emb0, emb1, sel0.reshape(B * C), sel1.reshape(B * C))

  return _attn(g0.reshape(B, C, D), g1.reshape(B, C, D), W_attn)
